# baseline (speedup 1.0000x reference)
import jax
import jax.numpy as jnp
from jax.experimental import pallas as pl

_BM = 400


def _sage_body(adj_ref, xb_ref, xi_ref, w1_ref, w2_ref, out_ref):
    acc = jnp.dot(adj_ref[...].astype(jnp.bfloat16), xb_ref[...],
                  preferred_element_type=jnp.float32)
    s = acc[:, 128:129]
    neigh = acc[:, :128] / (s + 1.0)
    out_ref[...] = jnp.maximum(
        jnp.dot(xi_ref[...], w1_ref[...], preferred_element_type=jnp.float32)
        + jnp.dot(neigh, w2_ref[...], preferred_element_type=jnp.float32),
        0.0,
    )


def kernel(x, adj, W):
    n, nfeat = x.shape
    nhid = W.shape[0]
    xb = jnp.concatenate(
        [x.astype(jnp.bfloat16),
         jnp.ones((n, 1), jnp.bfloat16),
         jnp.zeros((n, 127), jnp.bfloat16)],
        axis=1,
    )
    w1 = W[:, :nfeat].T
    w2 = W[:, nfeat:].T
    return pl.pallas_call(
        _sage_body,
        grid=(n // _BM,),
        in_specs=[
            pl.BlockSpec((_BM, n), lambda i: (i, 0)),
            pl.BlockSpec((n, 2 * nfeat), lambda i: (0, 0)),
            pl.BlockSpec((_BM, nfeat), lambda i: (i, 0)),
            pl.BlockSpec((nfeat, nhid), lambda i: (0, 0)),
            pl.BlockSpec((nfeat, nhid), lambda i: (0, 0)),
        ],
        out_specs=pl.BlockSpec((_BM, nhid), lambda i: (i, 0)),
        out_shape=jax.ShapeDtypeStruct((n, nhid), x.dtype),
    )(adj, xb, x, w1, w2)


# speedup vs baseline: 1.0366x; 1.0366x over previous
import jax
import jax.numpy as jnp
from jax.experimental import pallas as pl

_BM = 200


def _sage_body(adja_ref, adjb_ref, xb_ref, xi_ref, w1_ref, w2_ref, out_ref):
    xb = xb_ref[...]
    for half, ref in enumerate((adja_ref, adjb_ref)):
        a = ref[...]
        acc = jnp.dot(a.astype(jnp.bfloat16), xb,
                      preferred_element_type=jnp.float32)
        s = jnp.sum(a, axis=1, keepdims=True)
        neigh = acc / (s + 1.0)
        rows = pl.ds(half * _BM, _BM)
        out_ref[rows, :] = jnp.maximum(
            jnp.dot(xi_ref[rows, :], w1_ref[...],
                    preferred_element_type=jnp.float32)
            + jnp.dot(neigh, w2_ref[...], preferred_element_type=jnp.float32),
            0.0,
        )


def kernel(x, adj, W):
    n, nfeat = x.shape
    nhid = W.shape[0]
    xb16 = x.astype(jnp.bfloat16)
    w1 = W[:, :nfeat].T
    w2 = W[:, nfeat:].T
    return pl.pallas_call(
        _sage_body,
        grid=(n // (2 * _BM),),
        in_specs=[
            pl.BlockSpec((_BM, n), lambda i: (2 * i, 0)),
            pl.BlockSpec((_BM, n), lambda i: (2 * i + 1, 0)),
            pl.BlockSpec((n, nfeat), lambda i: (0, 0)),
            pl.BlockSpec((2 * _BM, nfeat), lambda i: (i, 0)),
            pl.BlockSpec((nfeat, nhid), lambda i: (0, 0)),
            pl.BlockSpec((nfeat, nhid), lambda i: (0, 0)),
        ],
        out_specs=pl.BlockSpec((2 * _BM, nhid), lambda i: (i, 0)),
        out_shape=jax.ShapeDtypeStruct((n, nhid), x.dtype),
    )(adj, adj, xb16, x, w1, w2)
